# baseline (speedup 1.0000x reference)
import functools

import jax
from jax import lax
import jax.numpy as jnp
from jax.experimental import pallas as pl
from jax.experimental.pallas import tpu as pltpu
from jax.experimental.pallas import tpu_sc as plsc


_EPS = 1e-12
_TOPK = 5


def _topk_body(q_ref, k_ref, out_i_ref, qn_bf, run_v, run_i, *, bk, nsteps):
    step = pl.program_id(0)

    @pl.when(step == 0)
    def _init():
        run_v[...] = jnp.full_like(run_v, -jnp.inf)
        run_i[...] = jnp.zeros_like(run_i)
        q = q_ref[...]
        qn = q / jnp.maximum(
            jnp.sqrt(jnp.sum(q * q, axis=1, keepdims=True)), _EPS
        )
        qn_bf[...] = qn.astype(jnp.bfloat16)

    kb = k_ref[...]
    kn = kb / jnp.maximum(
        jnp.sqrt(jnp.sum(kb * kb, axis=1, keepdims=True)), _EPS
    )
    simsT = jax.lax.dot_general(
        kn.astype(jnp.bfloat16),
        qn_bf[...],
        (((1,), (1,)), ((), ())),
        preferred_element_type=jnp.float32,
    )

    rowid = jax.lax.broadcasted_iota(jnp.int32, simsT.shape, 0) + step * bk
    big = jnp.int32(2147483647)

    blk_v = []
    blk_i = []
    excl = None
    for _ in range(_TOPK):
        s_eff = simsT if excl is None else jnp.where(excl, -jnp.inf, simsT)
        m = jnp.max(s_eff, axis=0, keepdims=True)
        am = jnp.min(
            jnp.where(s_eff == m, rowid, big), axis=0, keepdims=True
        )
        blk_v.append(m)
        blk_i.append(am)
        hit = rowid == am
        excl = hit if excl is None else excl | hit

    cand_v = jnp.concatenate([run_v[...]] + blk_v, axis=0)
    cand_i = jnp.concatenate([run_i[...]] + blk_i, axis=0)
    pos = jax.lax.broadcasted_iota(jnp.int32, cand_v.shape, 0)

    new_v = []
    new_i = []
    for _ in range(_TOPK):
        m = jnp.max(cand_v, axis=0, keepdims=True)
        p = jnp.min(jnp.where(cand_v == m, pos, big), axis=0, keepdims=True)
        sel = pos == p
        new_v.append(m)
        new_i.append(jnp.sum(jnp.where(sel, cand_i, 0), axis=0, keepdims=True))
        cand_v = jnp.where(sel, -jnp.inf, cand_v)

    run_v[...] = jnp.concatenate(new_v, axis=0)
    run_i[...] = jnp.concatenate(new_i, axis=0)

    @pl.when(step == nsteps - 1)
    def _fin():
        out_i_ref[...] = run_i[...]


def _topk_call(query, memory_keys):
    q, d = query.shape
    k, _ = memory_keys.shape
    bk = 2000
    if k % bk:
        bk = k
    nsteps = k // bk

    return pl.pallas_call(
        functools.partial(_topk_body, bk=bk, nsteps=nsteps),
        grid=(nsteps,),
        in_specs=[
            pl.BlockSpec((q, d), lambda i: (0, 0)),
            pl.BlockSpec((bk, d), lambda i: (i, 0)),
        ],
        out_specs=pl.BlockSpec((_TOPK, q), lambda i: (0, 0)),
        out_shape=jax.ShapeDtypeStruct((_TOPK, q), jnp.int32),
        scratch_shapes=[
            pltpu.VMEM((q, d), jnp.bfloat16),
            pltpu.VMEM((_TOPK, q), jnp.float32),
            pltpu.VMEM((_TOPK, q), jnp.int32),
        ],
    )(query, memory_keys)


def _sc_gather_call(memory_values, idx_flat):
    n, d = memory_values.shape
    b = idx_flat.shape[0]
    info = plsc.get_sparse_core_info()
    nw = info.num_cores * info.num_subcores
    b_per_w = b // nw
    nc = info.num_cores
    mesh = plsc.VectorSubcoreMesh(core_axis_name="c", subcore_axis_name="s")

    @functools.partial(
        pl.kernel,
        mesh=mesh,
        out_type=jax.ShapeDtypeStruct((b, d), jnp.float32),
        scratch_types=[
            pltpu.VMEM((b_per_w,), jnp.int32),
            pltpu.VMEM((b_per_w, d), jnp.float32),
            pltpu.SemaphoreType.DMA,
        ],
    )
    def gather_kernel(table_hbm, idx_hbm, out_hbm, idx_v, rows_v, sem):
        wid = lax.axis_index("s") * nc + lax.axis_index("c")
        base = wid * b_per_w
        pltpu.sync_copy(idx_hbm.at[pl.ds(base, b_per_w)], idx_v)
        pltpu.async_copy(table_hbm.at[idx_v], rows_v, sem).wait()
        pltpu.sync_copy(rows_v, out_hbm.at[pl.ds(base, b_per_w)])

    return gather_kernel(memory_values, idx_flat)


def _mean_body(g0, g1, g2, g3, g4, out_ref):
    out_ref[...] = (
        g0[...] + g1[...] + g2[...] + g3[...] + g4[...]
    ) * jnp.float32(0.2)


@jax.jit
def kernel(query, memory_keys, memory_values):
    q, d = query.shape
    k, _ = memory_values.shape

    top_idx = _topk_call(query, memory_keys)

    idx_flat = top_idx.reshape(_TOPK * q)
    gathered = _sc_gather_call(memory_values, idx_flat)

    mean_spec = lambda kk: pl.BlockSpec((q, d), lambda i: (kk, 0))
    retrieved = pl.pallas_call(
        _mean_body,
        grid=(1,),
        in_specs=[mean_spec(kk) for kk in range(_TOPK)],
        out_specs=pl.BlockSpec((q, d), lambda i: (0, 0)),
        out_shape=jax.ShapeDtypeStruct((q, d), jnp.float32),
    )(*([gathered] * _TOPK))

    return retrieved


# speedup vs baseline: 1.2394x; 1.2394x over previous
import functools

import jax
from jax import lax
import jax.numpy as jnp
from jax.experimental import pallas as pl
from jax.experimental.pallas import tpu as pltpu
from jax.experimental.pallas import tpu_sc as plsc


_EPS = 1e-12
_TOPK = 5


def _topk_body(q_ref, k_ref, out_i_ref, qn_bf, run_v, run_i, *, bk, nsteps):
    step = pl.program_id(0)

    @pl.when(step == 0)
    def _init():
        run_v[...] = jnp.full_like(run_v, -jnp.inf)
        run_i[...] = jnp.zeros_like(run_i)
        q = q_ref[...]
        qn = q / jnp.maximum(
            jnp.sqrt(jnp.sum(q * q, axis=1, keepdims=True)), _EPS
        )
        qn_bf[...] = qn.astype(jnp.bfloat16)

    kb = k_ref[...]
    kn = kb / jnp.maximum(
        jnp.sqrt(jnp.sum(kb * kb, axis=1, keepdims=True)), _EPS
    )
    simsT = jax.lax.dot_general(
        kn.astype(jnp.bfloat16),
        qn_bf[...],
        (((1,), (1,)), ((), ())),
        preferred_element_type=jnp.float32,
    )

    rowid = jax.lax.broadcasted_iota(jnp.int32, simsT.shape, 0) + step * bk
    big = jnp.int32(2147483647)

    blk_v = []
    blk_i = []
    for _ in range(_TOPK):
        m = jnp.max(simsT, axis=0, keepdims=True)
        am_l = jnp.argmax(simsT, axis=0).astype(jnp.int32).reshape(1, -1)
        am = am_l + step * bk
        blk_v.append(m)
        blk_i.append(am)
        simsT = jnp.where(rowid == am, -jnp.inf, simsT)

    cand_v = jnp.concatenate([run_v[...]] + blk_v, axis=0)
    cand_i = jnp.concatenate([run_i[...]] + blk_i, axis=0)
    pos = jax.lax.broadcasted_iota(jnp.int32, cand_v.shape, 0)

    new_v = []
    new_i = []
    for _ in range(_TOPK):
        m = jnp.max(cand_v, axis=0, keepdims=True)
        p = jnp.min(jnp.where(cand_v == m, pos, big), axis=0, keepdims=True)
        sel = pos == p
        new_v.append(m)
        new_i.append(jnp.sum(jnp.where(sel, cand_i, 0), axis=0, keepdims=True))
        cand_v = jnp.where(sel, -jnp.inf, cand_v)

    run_v[...] = jnp.concatenate(new_v, axis=0)
    run_i[...] = jnp.concatenate(new_i, axis=0)

    @pl.when(step == nsteps - 1)
    def _fin():
        out_i_ref[...] = run_i[...]


def _topk_call(query, memory_keys):
    q, d = query.shape
    k, _ = memory_keys.shape
    bk = 2000
    if k % bk:
        bk = k
    nsteps = k // bk

    return pl.pallas_call(
        functools.partial(_topk_body, bk=bk, nsteps=nsteps),
        grid=(nsteps,),
        in_specs=[
            pl.BlockSpec((q, d), lambda i: (0, 0)),
            pl.BlockSpec((bk, d), lambda i: (i, 0)),
        ],
        out_specs=pl.BlockSpec((_TOPK, q), lambda i: (0, 0)),
        out_shape=jax.ShapeDtypeStruct((_TOPK, q), jnp.int32),
        scratch_shapes=[
            pltpu.VMEM((q, d), jnp.bfloat16),
            pltpu.VMEM((_TOPK, q), jnp.float32),
            pltpu.VMEM((_TOPK, q), jnp.int32),
        ],
    )(query, memory_keys)


def _sc_gather_call(memory_values, idx_flat):
    n, d = memory_values.shape
    b = idx_flat.shape[0]
    info = plsc.get_sparse_core_info()
    nw = info.num_cores * info.num_subcores
    b_per_w = b // nw
    nc = info.num_cores
    mesh = plsc.VectorSubcoreMesh(core_axis_name="c", subcore_axis_name="s")

    @functools.partial(
        pl.kernel,
        mesh=mesh,
        out_type=jax.ShapeDtypeStruct((b, d), jnp.float32),
        scratch_types=[
            pltpu.VMEM((b_per_w,), jnp.int32),
            pltpu.VMEM((b_per_w, d), jnp.float32),
            pltpu.SemaphoreType.DMA,
        ],
    )
    def gather_kernel(table_hbm, idx_hbm, out_hbm, idx_v, rows_v, sem):
        wid = lax.axis_index("s") * nc + lax.axis_index("c")
        base = wid * b_per_w
        pltpu.sync_copy(idx_hbm.at[pl.ds(base, b_per_w)], idx_v)
        pltpu.async_copy(table_hbm.at[idx_v], rows_v, sem).wait()
        pltpu.sync_copy(rows_v, out_hbm.at[pl.ds(base, b_per_w)])

    return gather_kernel(memory_values, idx_flat)


def _mean_body(g0, g1, g2, g3, g4, out_ref):
    out_ref[...] = (
        g0[...] + g1[...] + g2[...] + g3[...] + g4[...]
    ) * jnp.float32(0.2)


@jax.jit
def kernel(query, memory_keys, memory_values):
    q, d = query.shape
    k, _ = memory_values.shape

    top_idx = _topk_call(query, memory_keys)

    idx_flat = top_idx.reshape(_TOPK * q)
    gathered = _sc_gather_call(memory_values, idx_flat)

    mean_spec = lambda kk: pl.BlockSpec((q, d), lambda i: (kk, 0))
    retrieved = pl.pallas_call(
        _mean_body,
        grid=(1,),
        in_specs=[mean_spec(kk) for kk in range(_TOPK)],
        out_specs=pl.BlockSpec((q, d), lambda i: (0, 0)),
        out_shape=jax.ShapeDtypeStruct((q, d), jnp.float32),
    )(*([gathered] * _TOPK))

    return retrieved


# speedup vs baseline: 1.2401x; 1.0005x over previous
import functools

import jax
from jax import lax
import jax.numpy as jnp
from jax.experimental import pallas as pl
from jax.experimental.pallas import tpu as pltpu
from jax.experimental.pallas import tpu_sc as plsc


_EPS = 1e-12
_TOPK = 5


def _topk_body(q_ref, k_ref, out_i_ref, qn_bf, run_v, run_i, *, bk, nsteps):
    step = pl.program_id(0)

    @pl.when(step == 0)
    def _init():
        run_v[...] = jnp.full_like(run_v, -jnp.inf)
        run_i[...] = jnp.zeros_like(run_i)
        q = q_ref[...]
        qn = q / jnp.maximum(
            jnp.sqrt(jnp.sum(q * q, axis=1, keepdims=True)), _EPS
        )
        qn_bf[...] = qn.astype(jnp.bfloat16)

    kb = k_ref[...]
    kn = kb / jnp.maximum(
        jnp.sqrt(jnp.sum(kb * kb, axis=1, keepdims=True)), _EPS
    )
    simsT = jax.lax.dot_general(
        kn.astype(jnp.bfloat16),
        qn_bf[...],
        (((1,), (1,)), ((), ())),
        preferred_element_type=jnp.float32,
    )

    rowid = jax.lax.broadcasted_iota(jnp.int32, simsT.shape, 0)

    blk_v = []
    blk_i = []
    for _ in range(_TOPK):
        am_l = jnp.argmax(simsT, axis=0).astype(jnp.int32).reshape(1, -1)
        m = jnp.max(simsT, axis=0, keepdims=True)
        blk_v.append(m)
        blk_i.append(am_l + step * bk)
        simsT = jnp.where(rowid == am_l, -jnp.inf, simsT)

    big = jnp.int32(2147483647)
    cand_v = jnp.concatenate([run_v[...]] + blk_v, axis=0)
    cand_i = jnp.concatenate([run_i[...]] + blk_i, axis=0)
    pos = jax.lax.broadcasted_iota(jnp.int32, cand_v.shape, 0)

    new_v = []
    new_i = []
    for _ in range(_TOPK):
        m = jnp.max(cand_v, axis=0, keepdims=True)
        p = jnp.min(jnp.where(cand_v == m, pos, big), axis=0, keepdims=True)
        sel = pos == p
        new_v.append(m)
        new_i.append(jnp.sum(jnp.where(sel, cand_i, 0), axis=0, keepdims=True))
        cand_v = jnp.where(sel, -jnp.inf, cand_v)

    run_v[...] = jnp.concatenate(new_v, axis=0)
    run_i[...] = jnp.concatenate(new_i, axis=0)

    @pl.when(step == nsteps - 1)
    def _fin():
        out_i_ref[...] = run_i[...]


def _topk_call(query, memory_keys):
    q, d = query.shape
    k, _ = memory_keys.shape
    bk = 2000
    if k % bk:
        bk = k
    nsteps = k // bk

    return pl.pallas_call(
        functools.partial(_topk_body, bk=bk, nsteps=nsteps),
        grid=(nsteps,),
        in_specs=[
            pl.BlockSpec((q, d), lambda i: (0, 0)),
            pl.BlockSpec((bk, d), lambda i: (i, 0)),
        ],
        out_specs=pl.BlockSpec((_TOPK, q), lambda i: (0, 0)),
        out_shape=jax.ShapeDtypeStruct((_TOPK, q), jnp.int32),
        scratch_shapes=[
            pltpu.VMEM((q, d), jnp.bfloat16),
            pltpu.VMEM((_TOPK, q), jnp.float32),
            pltpu.VMEM((_TOPK, q), jnp.int32),
        ],
    )(query, memory_keys)


def _sc_gather_call(memory_values, idx_flat):
    n, d = memory_values.shape
    b = idx_flat.shape[0]
    info = plsc.get_sparse_core_info()
    nw = info.num_cores * info.num_subcores
    b_per_w = b // nw
    nc = info.num_cores
    mesh = plsc.VectorSubcoreMesh(core_axis_name="c", subcore_axis_name="s")

    @functools.partial(
        pl.kernel,
        mesh=mesh,
        out_type=jax.ShapeDtypeStruct((b, d), jnp.float32),
        scratch_types=[
            pltpu.VMEM((b_per_w,), jnp.int32),
            pltpu.VMEM((b_per_w, d), jnp.float32),
            pltpu.SemaphoreType.DMA,
        ],
    )
    def gather_kernel(table_hbm, idx_hbm, out_hbm, idx_v, rows_v, sem):
        wid = lax.axis_index("s") * nc + lax.axis_index("c")
        base = wid * b_per_w
        pltpu.sync_copy(idx_hbm.at[pl.ds(base, b_per_w)], idx_v)
        pltpu.async_copy(table_hbm.at[idx_v], rows_v, sem).wait()
        pltpu.sync_copy(rows_v, out_hbm.at[pl.ds(base, b_per_w)])

    return gather_kernel(memory_values, idx_flat)


def _mean_body(g0, g1, g2, g3, g4, out_ref):
    out_ref[...] = (
        g0[...] + g1[...] + g2[...] + g3[...] + g4[...]
    ) * jnp.float32(0.2)


@jax.jit
def kernel(query, memory_keys, memory_values):
    q, d = query.shape
    k, _ = memory_values.shape

    top_idx = _topk_call(query, memory_keys)

    idx_flat = top_idx.reshape(_TOPK * q)
    gathered = _sc_gather_call(memory_values, idx_flat)

    mean_spec = lambda kk: pl.BlockSpec((q, d), lambda i: (kk, 0))
    retrieved = pl.pallas_call(
        _mean_body,
        grid=(1,),
        in_specs=[mean_spec(kk) for kk in range(_TOPK)],
        out_specs=pl.BlockSpec((q, d), lambda i: (0, 0)),
        out_shape=jax.ShapeDtypeStruct((q, d), jnp.float32),
    )(*([gathered] * _TOPK))

    return retrieved


# speedup vs baseline: 1.2631x; 1.0186x over previous
import functools

import jax
from jax import lax
import jax.numpy as jnp
from jax.experimental import pallas as pl
from jax.experimental.pallas import tpu as pltpu
from jax.experimental.pallas import tpu_sc as plsc


_EPS = 1e-12
_TOPK = 5


def _topk_body(q_ref, k_ref, out_i_ref, qn_bf, run_v, run_i, *, bk, nsteps):
    step = pl.program_id(0)

    @pl.when(step == 0)
    def _init():
        run_v[...] = jnp.full_like(run_v, -jnp.inf)
        run_i[...] = jnp.zeros_like(run_i)
        q = q_ref[...]
        qn = q / jnp.maximum(
            jnp.sqrt(jnp.sum(q * q, axis=1, keepdims=True)), _EPS
        )
        qn_bf[...] = qn.astype(jnp.bfloat16)

    kb = k_ref[...]
    kn = kb / jnp.maximum(
        jnp.sqrt(jnp.sum(kb * kb, axis=1, keepdims=True)), _EPS
    )
    simsT = jax.lax.dot_general(
        kn.astype(jnp.bfloat16),
        qn_bf[...],
        (((1,), (1,)), ((), ())),
        preferred_element_type=jnp.float32,
    )

    rowid = jax.lax.broadcasted_iota(jnp.int32, simsT.shape, 0)

    blk_v = []
    blk_i = []
    for _ in range(_TOPK):
        am_l = jnp.argmax(simsT, axis=0).astype(jnp.int32).reshape(1, -1)
        m = jnp.max(simsT, axis=0, keepdims=True)
        blk_v.append(m)
        blk_i.append(am_l + step * bk)
        simsT = jnp.where(rowid == am_l, -jnp.inf, simsT)

    big = jnp.int32(2147483647)
    cand_v = jnp.concatenate([run_v[...]] + blk_v, axis=0)
    cand_i = jnp.concatenate([run_i[...]] + blk_i, axis=0)
    pos = jax.lax.broadcasted_iota(jnp.int32, cand_v.shape, 0)

    new_v = []
    new_i = []
    for _ in range(_TOPK):
        m = jnp.max(cand_v, axis=0, keepdims=True)
        p = jnp.min(jnp.where(cand_v == m, pos, big), axis=0, keepdims=True)
        sel = pos == p
        new_v.append(m)
        new_i.append(jnp.sum(jnp.where(sel, cand_i, 0), axis=0, keepdims=True))
        cand_v = jnp.where(sel, -jnp.inf, cand_v)

    run_v[...] = jnp.concatenate(new_v, axis=0)
    run_i[...] = jnp.concatenate(new_i, axis=0)

    @pl.when(step == nsteps - 1)
    def _fin():
        out_i_ref[...] = run_i[...]


def _topk_call(query, memory_keys):
    q, d = query.shape
    k, _ = memory_keys.shape
    bk = 4000
    if k % bk:
        bk = k
    nsteps = k // bk

    return pl.pallas_call(
        functools.partial(_topk_body, bk=bk, nsteps=nsteps),
        grid=(nsteps,),
        in_specs=[
            pl.BlockSpec((q, d), lambda i: (0, 0)),
            pl.BlockSpec((bk, d), lambda i: (i, 0)),
        ],
        out_specs=pl.BlockSpec((_TOPK, q), lambda i: (0, 0)),
        out_shape=jax.ShapeDtypeStruct((_TOPK, q), jnp.int32),
        scratch_shapes=[
            pltpu.VMEM((q, d), jnp.bfloat16),
            pltpu.VMEM((_TOPK, q), jnp.float32),
            pltpu.VMEM((_TOPK, q), jnp.int32),
        ],
    )(query, memory_keys)


def _sc_gather_call(memory_values, idx_flat):
    n, d = memory_values.shape
    b = idx_flat.shape[0]
    info = plsc.get_sparse_core_info()
    nw = info.num_cores * info.num_subcores
    b_per_w = b // nw
    nc = info.num_cores
    mesh = plsc.VectorSubcoreMesh(core_axis_name="c", subcore_axis_name="s")

    @functools.partial(
        pl.kernel,
        mesh=mesh,
        out_type=jax.ShapeDtypeStruct((b, d), jnp.float32),
        scratch_types=[
            pltpu.VMEM((b_per_w,), jnp.int32),
            pltpu.VMEM((b_per_w, d), jnp.float32),
            pltpu.SemaphoreType.DMA,
        ],
    )
    def gather_kernel(table_hbm, idx_hbm, out_hbm, idx_v, rows_v, sem):
        wid = lax.axis_index("s") * nc + lax.axis_index("c")
        base = wid * b_per_w
        pltpu.sync_copy(idx_hbm.at[pl.ds(base, b_per_w)], idx_v)
        pltpu.async_copy(table_hbm.at[idx_v], rows_v, sem).wait()
        pltpu.sync_copy(rows_v, out_hbm.at[pl.ds(base, b_per_w)])

    return gather_kernel(memory_values, idx_flat)


def _mean_body(g0, g1, g2, g3, g4, out_ref):
    out_ref[...] = (
        g0[...] + g1[...] + g2[...] + g3[...] + g4[...]
    ) * jnp.float32(0.2)


@jax.jit
def kernel(query, memory_keys, memory_values):
    q, d = query.shape
    k, _ = memory_values.shape

    top_idx = _topk_call(query, memory_keys)

    idx_flat = top_idx.reshape(_TOPK * q)
    gathered = _sc_gather_call(memory_values, idx_flat)

    mean_spec = lambda kk: pl.BlockSpec((q, d), lambda i: (kk, 0))
    retrieved = pl.pallas_call(
        _mean_body,
        grid=(1,),
        in_specs=[mean_spec(kk) for kk in range(_TOPK)],
        out_specs=pl.BlockSpec((q, d), lambda i: (0, 0)),
        out_shape=jax.ShapeDtypeStruct((q, d), jnp.float32),
    )(*([gathered] * _TOPK))

    return retrieved
